# Initial kernel scaffold; baseline (speedup 1.0000x reference)
import jax
import jax.numpy as jnp
from jax.experimental import pallas as pl


def kernel(x, alphas, ctx_inp, W_prob, b_prob, W_gen, b_gen, gen_to_out, inp_to_out):
    raise NotImplementedError("write your pallas kernel here")


# speedup vs baseline: 148.3763x; 148.3763x over previous
import functools

import jax
import jax.numpy as jnp
from jax import lax
from jax.experimental import pallas as pl
from jax.experimental.pallas import tpu as pltpu
from jax.experimental.pallas import tpu_sc as plsc

B = 1024
D = 256
V = 30000
INP_V = 20000
S = 200
SPAD = 208
NC, NS, L = 2, 16, 16
NW = NC * NS
ROWS_PER_W = B // NW

TV = 2048
NT = 15

CHUNK = 128
N_FULL_CHUNKS = V // CHUNK
TAIL = V - N_FULL_CHUNKS * CHUNK
CHUNKS_PER_W = (N_FULL_CHUNKS + NW - 1) // NW

@functools.lru_cache(maxsize=None)
def _sc_mesh():
  return plsc.VectorSubcoreMesh(core_axis_name="c", subcore_axis_name="s",
                                num_cores=NC, num_subcores=NS)


def _bf16_as_f32(x):
  n, k2 = x.shape
  return lax.bitcast_convert_type(x.reshape(n, k2 // 2, 2), jnp.float32)


def _f32_as_bf16(x):
  n, k = x.shape
  return lax.bitcast_convert_type(x, jnp.bfloat16).reshape(n, 2 * k)


def _permute_body(wt_hbm, g2o_hbm, bg_hbm, wp_hbm, bp_hbm,
                  idx_v, rows_v, idx_t, rows_t, g2o_v, bg_v, bp_v):
  wid = lax.axis_index("s") * NC + lax.axis_index("c")

  def chunk(i, _):
    c = wid + i * NW

    @pl.when(c < N_FULL_CHUNKS)
    def _():
      base = c * CHUNK
      pltpu.sync_copy(g2o_hbm.at[pl.ds(base, CHUNK)], idx_v)
      pltpu.sync_copy(wt_hbm.at[pl.ds(base, CHUNK), :], rows_v)
      pltpu.sync_copy(rows_v, wp_hbm.at[idx_v])
    return 0

  lax.fori_loop(0, CHUNKS_PER_W, chunk, 0)

  @pl.when(wid == NW - 1)
  def _():
    base = N_FULL_CHUNKS * CHUNK
    pltpu.sync_copy(g2o_hbm.at[pl.ds(base, TAIL)], idx_t)
    pltpu.sync_copy(wt_hbm.at[pl.ds(base, TAIL), :], rows_t)
    pltpu.sync_copy(rows_t, wp_hbm.at[idx_t])

  @pl.when(wid == 0)
  def _():
    pltpu.sync_copy(g2o_hbm, g2o_v)
    pltpu.sync_copy(bg_hbm, bg_v)

    def scatter_bias(i, _):
      idx16 = g2o_v[pl.ds(i * L, L)]
      v16 = bg_v[pl.ds(i * L, L)]
      plsc.store_scatter(bp_v, [idx16], v16)
      return 0

    lax.fori_loop(0, V // L, scatter_bias, 0)
    pltpu.sync_copy(bp_v, bp_hbm)


@functools.lru_cache(maxsize=None)
def _permute_table():
  return pl.kernel(
    _permute_body,
    out_type=(jax.ShapeDtypeStruct((V, D // 2), jnp.float32),
              jax.ShapeDtypeStruct((V,), jnp.float32)),
    mesh=_sc_mesh(),
    compiler_params=pltpu.CompilerParams(needs_layout_passes=False),
    scratch_types=[
        pltpu.VMEM((CHUNK,), jnp.int32),
        pltpu.VMEM((CHUNK, D // 2), jnp.float32),
        pltpu.VMEM((TAIL,), jnp.int32),
        pltpu.VMEM((TAIL, D // 2), jnp.float32),
        pltpu.VMEM((V,), jnp.int32),
        pltpu.VMEM((V,), jnp.float32),
        pltpu.VMEM((V,), jnp.float32),
    ],
  )


def _ptr_body(ctx_hbm, al_hbm, i2o_hbm, ptr_hbm,
              i2o_v, row_v, ctx_v, a_v, cidx_v, shift_v):
  wid = lax.axis_index("s") * NC + lax.axis_index("c")
  pltpu.sync_copy(i2o_hbm, i2o_v)

  zeros16 = jnp.zeros((L,), jnp.float32)
  iota16 = lax.iota(jnp.int32, L)
  last_lane = iota16 == L - 1
  not_last = iota16 < L - 1
  shift_v[pl.ds(L, L)] = jnp.zeros((L,), jnp.int32)

  def zero_row(i, _):
    row_v[pl.ds(i * L, L)] = zeros16
    return 0

  lax.fori_loop(0, V // L, zero_row, 0)

  def do_row(j, _):
    b = wid * ROWS_PER_W + j
    pltpu.sync_copy(ctx_hbm.at[b], ctx_v)
    pltpu.sync_copy(al_hbm.at[b], a_v)

    def accum(k, _):
      i16 = ctx_v[pl.ds(k * L, L)]
      c16 = plsc.load_gather(i2o_v, [i16])
      cidx_v[pl.ds(k * L, L)] = c16
      v16 = a_v[pl.ds(k * L, L)]
      ks, vs = plsc.sort_key_val(c16, v16)
      cs = plsc.cumsum(vs)
      shift_v[pl.ds(0, L)] = ks
      nxt = shift_v[pl.ds(1, L)]
      segend = (ks != nxt) | last_lane
      plsc.addupdate_scatter(row_v, [ks], cs, mask=segend)
      plsc.addupdate_scatter(row_v, [nxt], -cs, mask=segend & not_last)
      return 0

    lax.fori_loop(0, SPAD // L, accum, 0)
    pltpu.sync_copy(row_v, ptr_hbm.at[b])

    def clear(k, _):
      c16 = cidx_v[pl.ds(k * L, L)]
      plsc.store_scatter(row_v, [c16], zeros16)
      return 0

    lax.fori_loop(0, SPAD // L, clear, 0)
    return 0

  lax.fori_loop(0, ROWS_PER_W, do_row, 0)


@functools.lru_cache(maxsize=None)
def _ptr_rows():
  return pl.kernel(
    _ptr_body,
    out_type=jax.ShapeDtypeStruct((B, V), jnp.float32),
    mesh=_sc_mesh(),
    compiler_params=pltpu.CompilerParams(needs_layout_passes=False),
    scratch_types=[
        pltpu.VMEM((INP_V,), jnp.int32),
        pltpu.VMEM((V,), jnp.float32),
        pltpu.VMEM((SPAD,), jnp.int32),
        pltpu.VMEM((SPAD,), jnp.float32),
        pltpu.VMEM((SPAD,), jnp.int32),
        pltpu.VMEM((2 * L,), jnp.int32),
    ],
  )


def _stats_body(x_r, wt_r, bg_r, wp_r, bpr_r, m_o, s_o, it_o, m_s, s_s):
  i = pl.program_id(0)

  @pl.when(i == 0)
  def _():
    m_s[:] = jnp.full((B, 1), -1e30, jnp.float32)
    s_s[:] = jnp.zeros((B, 1), jnp.float32)
    xw = jnp.dot(x_r[:], wp_r[:].astype(jnp.bfloat16),
                 preferred_element_type=jnp.float32)
    it_o[:] = jax.nn.sigmoid(xw + bpr_r[0, 0])

  l = lax.dot_general(x_r[:], wt_r[:], (((1,), (1,)), ((), ())),
                      preferred_element_type=jnp.float32)
  l = l + bg_r[:][0][None, :]
  col = lax.broadcasted_iota(jnp.int32, (B, TV), 1) + i * TV
  l = jnp.where(col < V, l, -1e30)
  tm = jnp.max(l, axis=1, keepdims=True)
  mnew = jnp.maximum(m_s[:], tm)
  s_s[:] = s_s[:] * jnp.exp(m_s[:] - mnew) + jnp.sum(
      jnp.exp(l - mnew), axis=1, keepdims=True)
  m_s[:] = mnew

  @pl.when(i == NT - 1)
  def _():
    m_o[:] = m_s[:]
    s_o[:] = s_s[:]


def _stats(x_bf, wt_bf, bg2d, w_prob, b_prob2d):
  return pl.pallas_call(
      _stats_body,
      grid=(NT,),
      in_specs=[
          pl.BlockSpec((B, D), lambda i: (0, 0)),
          pl.BlockSpec((TV, D), lambda i: (i, 0)),
          pl.BlockSpec((1, TV), lambda i: (0, i)),
          pl.BlockSpec((D, 1), lambda i: (0, 0)),
          pl.BlockSpec((1, 1), lambda i: (0, 0)),
      ],
      out_specs=[
          pl.BlockSpec((B, 1), lambda i: (0, 0)),
          pl.BlockSpec((B, 1), lambda i: (0, 0)),
          pl.BlockSpec((B, 1), lambda i: (0, 0)),
      ],
      out_shape=[
          jax.ShapeDtypeStruct((B, 1), jnp.float32),
          jax.ShapeDtypeStruct((B, 1), jnp.float32),
          jax.ShapeDtypeStruct((B, 1), jnp.float32),
      ],
      scratch_shapes=[
          pltpu.VMEM((B, 1), jnp.float32),
          pltpu.VMEM((B, 1), jnp.float32),
      ],
  )(x_bf, wt_bf, bg2d, w_prob, b_prob2d)


def _emit_body(x_r, wp_r, bp_r, m_r, s_r, it_r, ptr_r, out_r):
  l = lax.dot_general(x_r[:], wp_r[:], (((1,), (1,)), ((), ())),
                      preferred_element_type=jnp.float32)
  l = l + bp_r[:][0][None, :]
  g = jnp.exp(l - m_r[:])
  it = it_r[:]
  val = (it / s_r[:]) * g + (1.0 - it) * ptr_r[:]
  out_r[:] = jnp.log(val)


def _emit(x_bf, wp_bf, bp2d, m, s, it, ptr):
  return pl.pallas_call(
      _emit_body,
      grid=(NT,),
      in_specs=[
          pl.BlockSpec((B, D), lambda i: (0, 0)),
          pl.BlockSpec((TV, D), lambda i: (i, 0)),
          pl.BlockSpec((1, TV), lambda i: (0, i)),
          pl.BlockSpec((B, 1), lambda i: (0, 0)),
          pl.BlockSpec((B, 1), lambda i: (0, 0)),
          pl.BlockSpec((B, 1), lambda i: (0, 0)),
          pl.BlockSpec((B, TV), lambda i: (0, i)),
      ],
      out_specs=pl.BlockSpec((B, TV), lambda i: (0, i)),
      out_shape=jax.ShapeDtypeStruct((B, V), jnp.float32),
  )(x_bf, wp_bf, bp2d, m, s, it, ptr)


def kernel(x, alphas, ctx_inp, W_prob, b_prob, W_gen, b_gen, gen_to_out,
           inp_to_out):
  x_bf = x.astype(jnp.bfloat16)
  wt_bf = W_gen.T.astype(jnp.bfloat16)
  wt_v = _bf16_as_f32(wt_bf)
  g2o = gen_to_out.astype(jnp.int32)
  i2o = inp_to_out.astype(jnp.int32)
  ctx_pad = jnp.pad(ctx_inp.astype(jnp.int32), ((0, 0), (0, SPAD - S)))
  al_pad = jnp.pad(alphas, ((0, 0), (0, SPAD - S)))

  wp_v, bp = _permute_table()(wt_v, g2o, b_gen)
  ptr = _ptr_rows()(ctx_pad, al_pad, i2o)
  m, s, it = _stats(x_bf, wt_bf, b_gen.reshape(1, V),
                    W_prob, b_prob.reshape(1, 1))
  wp_bf = _f32_as_bf16(wp_v)
  out = _emit(x_bf, wp_bf, bp.reshape(1, V), m, s, it, ptr)
  return out


# speedup vs baseline: 155.9950x; 1.0513x over previous
import functools

import jax
import jax.numpy as jnp
from jax import lax
from jax.experimental import pallas as pl
from jax.experimental.pallas import tpu as pltpu
from jax.experimental.pallas import tpu_sc as plsc

B = 1024
D = 256
DV = D // 2
V = 30000
INP_V = 20000
S = 200
NC, NS, L = 2, 16, 16
NW = NC * NS
ROWS_PER_W = B // NW
CTX_PER_W = ROWS_PER_W * S

TV = 2048
NT = 15

CHUNK = 128
N_FULL_CHUNKS = V // CHUNK
TAIL = V - N_FULL_CHUNKS * CHUNK
CHUNKS_PER_W = (N_FULL_CHUNKS + NW - 1) // NW


@functools.lru_cache(maxsize=None)
def _sc_mesh():
  return plsc.VectorSubcoreMesh(core_axis_name="c", subcore_axis_name="s",
                                num_cores=NC, num_subcores=NS)


def _bf16_as_f32(x):
  n, k2 = x.shape
  return lax.bitcast_convert_type(x.reshape(n, k2 // 2, 2), jnp.float32)


def _f32_as_bf16(x):
  n, k = x.shape
  return lax.bitcast_convert_type(x, jnp.bfloat16).reshape(n, 2 * k)


def _permute_body(wt_hbm, g2o_hbm, bg_hbm, wp_hbm, bp_hbm,
                  idx_v, rows_v, idx_t, rows_t, g2o_v, bg_v, bp_v):
  wid = lax.axis_index("s") * NC + lax.axis_index("c")

  def chunk(i, _):
    c = wid + i * NW

    @pl.when(c < N_FULL_CHUNKS)
    def _():
      base = c * CHUNK
      pltpu.sync_copy(g2o_hbm.at[pl.ds(base, CHUNK)], idx_v)
      pltpu.sync_copy(wt_hbm.at[pl.ds(base, CHUNK), :], rows_v)
      pltpu.sync_copy(rows_v, wp_hbm.at[idx_v])
    return 0

  lax.fori_loop(0, CHUNKS_PER_W, chunk, 0)

  @pl.when(wid == NW - 1)
  def _():
    base = N_FULL_CHUNKS * CHUNK
    pltpu.sync_copy(g2o_hbm.at[pl.ds(base, TAIL)], idx_t)
    pltpu.sync_copy(wt_hbm.at[pl.ds(base, TAIL), :], rows_t)
    pltpu.sync_copy(rows_t, wp_hbm.at[idx_t])

  @pl.when(wid == 0)
  def _():
    pltpu.sync_copy(g2o_hbm, g2o_v)
    pltpu.sync_copy(bg_hbm, bg_v)

    def scatter_bias(i, _):
      idx16 = g2o_v[pl.ds(i * L, L)]
      v16 = bg_v[pl.ds(i * L, L)]
      plsc.store_scatter(bp_v, [idx16], v16)
      return 0

    lax.fori_loop(0, V // L, scatter_bias, 0)
    pltpu.sync_copy(bp_v, bp_hbm)


@functools.lru_cache(maxsize=None)
def _permute_table():
  return pl.kernel(
      _permute_body,
      out_type=(jax.ShapeDtypeStruct((V, DV), jnp.float32),
                jax.ShapeDtypeStruct((V,), jnp.float32)),
      mesh=_sc_mesh(),
      compiler_params=pltpu.CompilerParams(needs_layout_passes=False),
      scratch_types=[
          pltpu.VMEM((CHUNK,), jnp.int32),
          pltpu.VMEM((CHUNK, DV), jnp.float32),
          pltpu.VMEM((TAIL,), jnp.int32),
          pltpu.VMEM((TAIL, DV), jnp.float32),
          pltpu.VMEM((V,), jnp.int32),
          pltpu.VMEM((V,), jnp.float32),
          pltpu.VMEM((V,), jnp.float32),
      ],
  )


def _ptr_body(ctx_hbm, al_hbm, i2o_hbm, ptr_hbm,
              i2o_v, ctx_v, al_v, row0_v, row1_v, cidx0_v, cidx1_v,
              shift_v, sem0, sem1):
  wid = lax.axis_index("s") * NC + lax.axis_index("c")

  pltpu.sync_copy(i2o_hbm, i2o_v)
  pltpu.sync_copy(ctx_hbm.at[pl.ds(wid * CTX_PER_W, CTX_PER_W)],
                  ctx_v.at[pl.ds(0, CTX_PER_W)])
  pltpu.sync_copy(al_hbm.at[pl.ds(wid * CTX_PER_W, CTX_PER_W)],
                  al_v.at[pl.ds(0, CTX_PER_W)])

  zeros16 = jnp.zeros((L,), jnp.float32)
  iota16 = lax.iota(jnp.int32, L)
  last_lane = iota16 == L - 1
  not_last = iota16 < L - 1
  shift_v[pl.ds(L, L)] = jnp.zeros((L,), jnp.int32)

  def zero_rows(i, _):
    row0_v[pl.ds(i * L, L)] = zeros16
    row1_v[pl.ds(i * L, L)] = zeros16
    return 0

  lax.fori_loop(0, V // L, zero_rows, 0)

  bufs = ((row0_v, cidx0_v, sem0), (row1_v, cidx1_v, sem1))
  nchunk = (S + L - 1) // L

  def pair(t, _):
    for p in (0, 1):
      row_v, cidx_v, sem = bufs[p]
      j = 2 * t + p
      b = wid * ROWS_PER_W + j

      @pl.when(t > 0)
      def _():
        pltpu.make_async_copy(row_v, ptr_hbm.at[b - 2], sem).wait()

        def clear(k, _):
          c16 = cidx_v[pl.ds(k * L, L)]
          plsc.store_scatter(row_v, [c16], zeros16)
          return 0

        lax.fori_loop(0, nchunk, clear, 0)

      def accum(k, _):
        off = j * S + k * L
        i16 = ctx_v[pl.ds(off, L)]
        i16 = jnp.clip(i16, 0, INP_V - 1)
        c16 = plsc.load_gather(i2o_v, [i16])
        cidx_v[pl.ds(k * L, L)] = c16
        v16 = al_v[pl.ds(off, L)]
        v16 = jnp.where(k * L + iota16 < S, v16, 0.0)
        ks, vs = plsc.sort_key_val(c16, v16)
        cs = plsc.cumsum(vs)
        shift_v[pl.ds(0, L)] = ks
        nxt = shift_v[pl.ds(1, L)]
        segend = (ks != nxt) | last_lane
        plsc.addupdate_scatter(row_v, [ks], cs, mask=segend)
        plsc.addupdate_scatter(row_v, [nxt], -cs, mask=segend & not_last)
        return 0

      lax.fori_loop(0, nchunk, accum, 0)
      pltpu.async_copy(row_v, ptr_hbm.at[b], sem)
    return 0

  lax.fori_loop(0, ROWS_PER_W // 2, pair, 0)
  last = wid * ROWS_PER_W + ROWS_PER_W - 2
  pltpu.make_async_copy(row0_v, ptr_hbm.at[last], sem0).wait()
  pltpu.make_async_copy(row1_v, ptr_hbm.at[last + 1], sem1).wait()


@functools.lru_cache(maxsize=None)
def _ptr_rows():
  return pl.kernel(
      _ptr_body,
      out_type=jax.ShapeDtypeStruct((B, V), jnp.float32),
      mesh=_sc_mesh(),
      compiler_params=pltpu.CompilerParams(needs_layout_passes=False),
      scratch_types=[
          pltpu.VMEM((INP_V,), jnp.int32),
          pltpu.VMEM((CTX_PER_W + 8,), jnp.int32),
          pltpu.VMEM((CTX_PER_W + 8,), jnp.float32),
          pltpu.VMEM((V,), jnp.float32),
          pltpu.VMEM((V,), jnp.float32),
          pltpu.VMEM((L * ((S + L - 1) // L),), jnp.int32),
          pltpu.VMEM((L * ((S + L - 1) // L),), jnp.int32),
          pltpu.VMEM((2 * L,), jnp.int32),
          pltpu.SemaphoreType.DMA,
          pltpu.SemaphoreType.DMA,
      ],
  )


def _stats_body(x_r, wt_r, bg_r, wp_r, bpr_r, m_o, s_o, it_o, m_s, s_s):
  i = pl.program_id(0)

  @pl.when(i == 0)
  def _():
    m_s[:] = jnp.full((B, 1), -1e30, jnp.float32)
    s_s[:] = jnp.zeros((B, 1), jnp.float32)
    xw = jnp.dot(x_r[:], wp_r[:].astype(jnp.bfloat16),
                 preferred_element_type=jnp.float32)
    it_o[:] = jax.nn.sigmoid(xw + bpr_r[0, 0])

  l = lax.dot_general(x_r[:], wt_r[:], (((1,), (1,)), ((), ())),
                      preferred_element_type=jnp.float32)
  l = l + bg_r[:][0][None, :]
  col = lax.broadcasted_iota(jnp.int32, (B, TV), 1) + i * TV
  l = jnp.where(col < V, l, -1e30)
  tm = jnp.max(l, axis=1, keepdims=True)
  mnew = jnp.maximum(m_s[:], tm)
  s_s[:] = s_s[:] * jnp.exp(m_s[:] - mnew) + jnp.sum(
      jnp.exp(l - mnew), axis=1, keepdims=True)
  m_s[:] = mnew

  @pl.when(i == NT - 1)
  def _():
    m_o[:] = m_s[:]
    s_o[:] = s_s[:]


def _stats(x_bf, wt_bf, bg2d, w_prob, b_prob2d):
  return pl.pallas_call(
      _stats_body,
      grid=(NT,),
      in_specs=[
          pl.BlockSpec((B, D), lambda i: (0, 0)),
          pl.BlockSpec((TV, D), lambda i: (i, 0)),
          pl.BlockSpec((1, TV), lambda i: (0, i)),
          pl.BlockSpec((D, 1), lambda i: (0, 0)),
          pl.BlockSpec((1, 1), lambda i: (0, 0)),
      ],
      out_specs=[
          pl.BlockSpec((B, 1), lambda i: (0, 0)),
          pl.BlockSpec((B, 1), lambda i: (0, 0)),
          pl.BlockSpec((B, 1), lambda i: (0, 0)),
      ],
      out_shape=[
          jax.ShapeDtypeStruct((B, 1), jnp.float32),
          jax.ShapeDtypeStruct((B, 1), jnp.float32),
          jax.ShapeDtypeStruct((B, 1), jnp.float32),
      ],
      scratch_shapes=[
          pltpu.VMEM((B, 1), jnp.float32),
          pltpu.VMEM((B, 1), jnp.float32),
      ],
  )(x_bf, wt_bf, bg2d, w_prob, b_prob2d)


def _emit_body(x_r, wp_r, bp_r, m_r, s_r, it_r, ptr_r, out_r):
  l = lax.dot_general(x_r[:], wp_r[:], (((1,), (1,)), ((), ())),
                      preferred_element_type=jnp.float32)
  l = l + bp_r[:][0][None, :]
  g = jnp.exp(l - m_r[:])
  it = it_r[:]
  val = (it / s_r[:]) * g + (1.0 - it) * ptr_r[:]
  out_r[:] = jnp.log(val)


def _emit(x_bf, wp_bf, bp2d, m, s, it, ptr):
  return pl.pallas_call(
      _emit_body,
      grid=(NT,),
      in_specs=[
          pl.BlockSpec((B, D), lambda i: (0, 0)),
          pl.BlockSpec((TV, D), lambda i: (i, 0)),
          pl.BlockSpec((1, TV), lambda i: (0, i)),
          pl.BlockSpec((B, 1), lambda i: (0, 0)),
          pl.BlockSpec((B, 1), lambda i: (0, 0)),
          pl.BlockSpec((B, 1), lambda i: (0, 0)),
          pl.BlockSpec((B, TV), lambda i: (0, i)),
      ],
      out_specs=pl.BlockSpec((B, TV), lambda i: (0, i)),
      out_shape=jax.ShapeDtypeStruct((B, V), jnp.float32),
  )(x_bf, wp_bf, bp2d, m, s, it, ptr)


def kernel(x, alphas, ctx_inp, W_prob, b_prob, W_gen, b_gen, gen_to_out,
           inp_to_out):
  x_bf = x.astype(jnp.bfloat16)
  wt_bf = W_gen.T.astype(jnp.bfloat16)
  wt_v = _bf16_as_f32(wt_bf)
  g2o = gen_to_out.astype(jnp.int32)
  i2o = inp_to_out.astype(jnp.int32)
  ctx_flat = ctx_inp.astype(jnp.int32).reshape(-1)
  al_flat = alphas.reshape(-1)

  wp_v, bp = _permute_table()(wt_v, g2o, b_gen)
  ptr = _ptr_rows()(ctx_flat, al_flat, i2o)
  m, s, it = _stats(x_bf, wt_bf, b_gen.reshape(1, V),
                    W_prob, b_prob.reshape(1, 1))
  wp_bf = _f32_as_bf16(wp_v)
  out = _emit(x_bf, wp_bf, bp.reshape(1, V), m, s, it, ptr)
  return out


# speedup vs baseline: 551.5169x; 3.5355x over previous
import functools

import jax
import jax.numpy as jnp
from jax import lax
from jax.experimental import pallas as pl
from jax.experimental.pallas import tpu as pltpu
from jax.experimental.pallas import tpu_sc as plsc

B = 1024
D = 256
V = 30000
INP_V = 20000
S = 200
NC, NS, L = 2, 16, 16
NW = NC * NS
ROWS_PER_W = B // NW
CTX_PER_W = ROWS_PER_W * S

TV = 2048
NT = 15

CHUNK = 128
N_FULL_CHUNKS = V // CHUNK
TAIL = V - N_FULL_CHUNKS * CHUNK
CHUNKS_PER_W = (N_FULL_CHUNKS + NW - 1) // NW


@functools.lru_cache(maxsize=None)
def _sc_mesh():
  return plsc.VectorSubcoreMesh(core_axis_name="c", subcore_axis_name="s",
                                num_cores=NC, num_subcores=NS)


def _permute_body(wt_hbm, g2o_hbm, bg_hbm, wp_hbm, bp_hbm,
                  idx_v, rows_v, idx_t, g2o_v, bg_v, bp_v):
  wid = lax.axis_index("s") * NC + lax.axis_index("c")

  def chunk(i, _):
    c = wid + i * NW

    @pl.when(c < N_FULL_CHUNKS)
    def _():
      base = c * CHUNK
      pltpu.sync_copy(g2o_hbm.at[pl.ds(base, CHUNK)], idx_v)
      pltpu.sync_copy(wt_hbm.at[pl.ds(base, CHUNK), :], rows_v)
      pltpu.sync_copy(rows_v, wp_hbm.at[idx_v])
    return 0

  lax.fori_loop(0, CHUNKS_PER_W, chunk, 0)

  @pl.when(wid == NW - 1)
  def _():
    base = N_FULL_CHUNKS * CHUNK
    pltpu.sync_copy(g2o_hbm.at[pl.ds(base, TAIL)], idx_t)
    pltpu.sync_copy(wt_hbm.at[pl.ds(base, TAIL), :],
                    rows_v.at[pl.ds(0, TAIL), :])
    pltpu.sync_copy(rows_v.at[pl.ds(0, TAIL), :], wp_hbm.at[idx_t])

  @pl.when(wid == 0)
  def _():
    pltpu.sync_copy(g2o_hbm, g2o_v)
    pltpu.sync_copy(bg_hbm, bg_v)

    def scatter_bias(i, _):
      idx16 = g2o_v[pl.ds(i * L, L)]
      v16 = bg_v[pl.ds(i * L, L)]
      plsc.store_scatter(bp_v, [idx16], v16)
      return 0

    lax.fori_loop(0, V // L, scatter_bias, 0)
    pltpu.sync_copy(bp_v, bp_hbm)


@functools.lru_cache(maxsize=None)
def _permute_table():
  return pl.kernel(
      _permute_body,
      out_type=(jax.ShapeDtypeStruct((V, D), jnp.float32),
                jax.ShapeDtypeStruct((V,), jnp.float32)),
      mesh=_sc_mesh(),
      compiler_params=pltpu.CompilerParams(needs_layout_passes=False),
      scratch_types=[
          pltpu.VMEM((CHUNK,), jnp.int32),
          pltpu.VMEM((CHUNK, D), jnp.float32),
          pltpu.VMEM((TAIL,), jnp.int32),
          pltpu.VMEM((V,), jnp.int32),
          pltpu.VMEM((V,), jnp.float32),
          pltpu.VMEM((V,), jnp.float32),
      ],
  )


def _ptr_body(ctx_hbm, al_hbm, i2o_hbm, ptr_hbm,
              i2o_v, ctx_v, al_v, row0_v, row1_v, cidx0_v, cidx1_v,
              shift_v, sem0, sem1):
  wid = lax.axis_index("s") * NC + lax.axis_index("c")

  pltpu.sync_copy(i2o_hbm, i2o_v)
  pltpu.sync_copy(ctx_hbm.at[pl.ds(wid * CTX_PER_W, CTX_PER_W)],
                  ctx_v.at[pl.ds(0, CTX_PER_W)])
  pltpu.sync_copy(al_hbm.at[pl.ds(wid * CTX_PER_W, CTX_PER_W)],
                  al_v.at[pl.ds(0, CTX_PER_W)])

  zeros16 = jnp.zeros((L,), jnp.float32)
  iota16 = lax.iota(jnp.int32, L)
  last_lane = iota16 == L - 1
  not_last = iota16 < L - 1
  shift_v[pl.ds(L, L)] = jnp.zeros((L,), jnp.int32)

  def zero_rows(i, _):
    row0_v[pl.ds(i * L, L)] = zeros16
    row1_v[pl.ds(i * L, L)] = zeros16
    return 0

  lax.fori_loop(0, V // L, zero_rows, 0)

  bufs = ((row0_v, cidx0_v, sem0), (row1_v, cidx1_v, sem1))
  nchunk = (S + L - 1) // L

  def pair(t, _):
    for p in (0, 1):
      row_v, cidx_v, sem = bufs[p]
      j = 2 * t + p
      b = wid * ROWS_PER_W + j

      @pl.when(t > 0)
      def _():
        pltpu.make_async_copy(row_v, ptr_hbm.at[b - 2], sem).wait()

        def clear(k, _):
          c16 = cidx_v[pl.ds(k * L, L)]
          plsc.store_scatter(row_v, [c16], zeros16)
          return 0

        lax.fori_loop(0, nchunk, clear, 0)

      def accum(k, _):
        off = j * S + k * L
        i16 = ctx_v[pl.ds(off, L)]
        i16 = jnp.clip(i16, 0, INP_V - 1)
        c16 = plsc.load_gather(i2o_v, [i16])
        cidx_v[pl.ds(k * L, L)] = c16
        v16 = al_v[pl.ds(off, L)]
        v16 = jnp.where(k * L + iota16 < S, v16, 0.0)
        ks, vs = plsc.sort_key_val(c16, v16)
        cs = plsc.cumsum(vs)
        shift_v[pl.ds(0, L)] = ks
        nxt = shift_v[pl.ds(1, L)]
        segend = (ks != nxt) | last_lane
        plsc.addupdate_scatter(row_v, [ks], cs, mask=segend)
        plsc.addupdate_scatter(row_v, [nxt], -cs, mask=segend & not_last)
        return 0

      lax.fori_loop(0, nchunk, accum, 0)
      pltpu.async_copy(row_v, ptr_hbm.at[b], sem)
    return 0

  lax.fori_loop(0, ROWS_PER_W // 2, pair, 0)
  last = wid * ROWS_PER_W + ROWS_PER_W - 2
  pltpu.make_async_copy(row0_v, ptr_hbm.at[last], sem0).wait()
  pltpu.make_async_copy(row1_v, ptr_hbm.at[last + 1], sem1).wait()


@functools.lru_cache(maxsize=None)
def _ptr_rows():
  return pl.kernel(
      _ptr_body,
      out_type=jax.ShapeDtypeStruct((B, V), jnp.float32),
      mesh=_sc_mesh(),
      compiler_params=pltpu.CompilerParams(needs_layout_passes=False),
      scratch_types=[
          pltpu.VMEM((INP_V,), jnp.int32),
          pltpu.VMEM((CTX_PER_W + 8,), jnp.int32),
          pltpu.VMEM((CTX_PER_W + 8,), jnp.float32),
          pltpu.VMEM((V,), jnp.float32),
          pltpu.VMEM((V,), jnp.float32),
          pltpu.VMEM((L * ((S + L - 1) // L),), jnp.int32),
          pltpu.VMEM((L * ((S + L - 1) // L),), jnp.int32),
          pltpu.VMEM((2 * L,), jnp.int32),
          pltpu.SemaphoreType.DMA,
          pltpu.SemaphoreType.DMA,
      ],
  )


def _stats_body(x_r, wt_r, bg_r, wpr_r, bpr_r, m_o, s_o, it_o, m_s, s_s):
  i = pl.program_id(0)

  @pl.when(i == 0)
  def _():
    m_s[:] = jnp.full((1, B), -1e30, jnp.float32)
    s_s[:] = jnp.zeros((1, B), jnp.float32)
    xw = lax.dot_general(wpr_r[:].astype(jnp.bfloat16), x_r[:],
                         (((0,), (1,)), ((), ())),
                         preferred_element_type=jnp.float32)
    it_o[:] = jax.nn.sigmoid(xw + bpr_r[0, 0])

  w = wt_r[:].astype(jnp.bfloat16)
  l = lax.dot_general(w, x_r[:], (((1,), (1,)), ((), ())),
                      preferred_element_type=jnp.float32)
  l = l + bg_r[:][0][:, None]
  row = lax.broadcasted_iota(jnp.int32, (TV, B), 0) + i * TV
  l = jnp.where(row < V, l, -1e30)
  tm = jnp.max(l, axis=0, keepdims=True)
  mnew = jnp.maximum(m_s[:], tm)
  s_s[:] = s_s[:] * jnp.exp(m_s[:] - mnew) + jnp.sum(
      jnp.exp(l - mnew), axis=0, keepdims=True)
  m_s[:] = mnew

  @pl.when(i == NT - 1)
  def _():
    m_o[:] = m_s[:]
    s_o[:] = s_s[:]


def _stats(x_bf, wt_f, bg2d, w_prob, b_prob2d):
  return pl.pallas_call(
      _stats_body,
      grid=(NT,),
      in_specs=[
          pl.BlockSpec((B, D), lambda i: (0, 0)),
          pl.BlockSpec((TV, D), lambda i: (i, 0)),
          pl.BlockSpec((1, TV), lambda i: (0, i)),
          pl.BlockSpec((D, 1), lambda i: (0, 0)),
          pl.BlockSpec((1, 1), lambda i: (0, 0)),
      ],
      out_specs=[
          pl.BlockSpec((1, B), lambda i: (0, 0)),
          pl.BlockSpec((1, B), lambda i: (0, 0)),
          pl.BlockSpec((1, B), lambda i: (0, 0)),
      ],
      out_shape=[
          jax.ShapeDtypeStruct((1, B), jnp.float32),
          jax.ShapeDtypeStruct((1, B), jnp.float32),
          jax.ShapeDtypeStruct((1, B), jnp.float32),
      ],
      scratch_shapes=[
          pltpu.VMEM((1, B), jnp.float32),
          pltpu.VMEM((1, B), jnp.float32),
      ],
  )(x_bf, wt_f, bg2d, w_prob, b_prob2d)


def _emit_body(x_r, wp_r, bp_r, m_r, s_r, it_r, ptr_r, out_r):
  w = wp_r[:].astype(jnp.bfloat16)
  l = lax.dot_general(w, x_r[:], (((1,), (1,)), ((), ())),
                      preferred_element_type=jnp.float32)
  l = l + bp_r[:][0][:, None]
  g = jnp.exp(l - m_r[:])
  it = it_r[:]
  p_t = lax.transpose(ptr_r[:], (1, 0))
  val = (it / s_r[:]) * g + (1.0 - it) * p_t
  out_r[:] = jnp.log(val)


def _emit(x_bf, wp_f, bp2d, m, s, it, ptr):
  return pl.pallas_call(
      _emit_body,
      grid=(NT,),
      in_specs=[
          pl.BlockSpec((B, D), lambda i: (0, 0)),
          pl.BlockSpec((TV, D), lambda i: (i, 0)),
          pl.BlockSpec((1, TV), lambda i: (0, i)),
          pl.BlockSpec((1, B), lambda i: (0, 0)),
          pl.BlockSpec((1, B), lambda i: (0, 0)),
          pl.BlockSpec((1, B), lambda i: (0, 0)),
          pl.BlockSpec((B, TV), lambda i: (0, i)),
      ],
      out_specs=pl.BlockSpec((TV, B), lambda i: (i, 0)),
      out_shape=jax.ShapeDtypeStruct((V, B), jnp.float32),
  )(x_bf, wp_f, bp2d, m, s, it, ptr)


def kernel(x, alphas, ctx_inp, W_prob, b_prob, W_gen, b_gen, gen_to_out,
           inp_to_out):
  x_bf = x.astype(jnp.bfloat16)
  wt_f = W_gen.T
  g2o = gen_to_out.astype(jnp.int32)
  i2o = inp_to_out.astype(jnp.int32)
  ctx_flat = ctx_inp.astype(jnp.int32).reshape(-1)
  al_flat = alphas.reshape(-1)

  wp_f, bp = _permute_table()(wt_f, g2o, b_gen)
  ptr = _ptr_rows()(ctx_flat, al_flat, i2o)
  m, s, it = _stats(x_bf, wt_f, b_gen.reshape(1, V),
                    W_prob, b_prob.reshape(1, 1))
  out_t = _emit(x_bf, wp_f, bp.reshape(1, V), m, s, it, ptr)
  return out_t.T
